# Initial kernel scaffold; baseline (speedup 1.0000x reference)
import functools

import jax
import jax.numpy as jnp
from jax import lax
from jax.experimental import pallas as pl
from jax.experimental.pallas import tpu as pltpu
from jax.experimental.pallas import tpu_sc as plsc

D = 16
NC = 2
NS = 16
NW = NC * NS
EPR = 128
UNROLL = 8
ZCH = 800


def _feat_body(x_ref, tfm_ref, o_ref, *, nk, nj):
    xb = x_ref[...]
    tfm = tfm_ref[...]
    tsq = jnp.sum(tfm * tfm, axis=0, keepdims=True)
    y = tsq - 2.0 * jnp.dot(xb, tfm, preferred_element_type=jnp.float32)
    m = y[:, 0:nk]
    for j in range(1, nj):
        m = jnp.minimum(m, y[:, j * nk:(j + 1) * nk])
    fd = m + jnp.sum(xb * xb, axis=1, keepdims=True)
    bsz = fd.shape[0]
    ones = jnp.ones((bsz, 1), jnp.float32)
    zeros = jnp.zeros((bsz, D - nk - 1), jnp.float32)
    o_ref[...] = jnp.concatenate([fd, ones, zeros], axis=1)


def _sc_body(src_hbm, dst_hbm, table_hbm, zeros_hbm, out_hbm,
             src_v, dst_v, rows_v, zbuf_v, agg_sh, sem,
             *, rpt, s_rpt):
    c = lax.axis_index("c")
    s = lax.axis_index("s")
    wid = c * NS + s

    pltpu.sync_copy(zeros_hbm, zbuf_v)
    my0 = s * s_rpt

    def _zero(k, _):
        pltpu.sync_copy(zbuf_v, agg_sh.at[pl.ds(my0 + k * ZCH, ZCH)])
        return 0

    lax.fori_loop(0, s_rpt // ZCH, _zero, 0)
    plsc.subcore_barrier()

    row0 = wid * rpt

    def _step(g, _):
        base = row0 + g * UNROLL
        pltpu.sync_copy(src_hbm.at[pl.ds(base, UNROLL)], src_v)
        pltpu.sync_copy(dst_hbm.at[pl.ds(base, UNROLL)], dst_v)
        for j in range(UNROLL):
            pltpu.async_copy(table_hbm.at[src_v.at[j]], rows_v, sem).wait()
            pltpu.sync_copy(rows_v, agg_sh.at[dst_v.at[j]], add=True)
        return 0

    lax.fori_loop(0, rpt // UNROLL, _step, 0)
    plsc.subcore_barrier()

    def _emit(k, _):
        r = my0 + k * ZCH
        pltpu.sync_copy(agg_sh.at[pl.ds(r, ZCH)], zbuf_v)
        pltpu.sync_copy(zbuf_v, out_hbm.at[c, pl.ds(r, ZCH)])
        return 0

    lax.fori_loop(0, s_rpt // ZCH, _emit, 0)


def _final_body(tab_ref, p_ref, tmat_ref, wt_ref, b_ref, o_ref, *, nk):
    a = p_ref[0] + p_ref[1]
    deg = jnp.maximum(a[:, nk:nk + 1], 1.0)
    nm = a[:, 0:nk] / deg
    fd = tab_ref[...][:, 0:nk]
    tmat = tmat_ref[...]
    ones = jnp.ones((1, tmat.shape[0]), jnp.float32)
    struct = jnp.dot(ones, tmat, preferred_element_type=jnp.float32)
    struct = struct * (1.0 / tmat.shape[0])
    h = 0.5 * (fd + nm) + struct
    o_ref[...] = (jnp.dot(h, wt_ref[...], preferred_element_type=jnp.float32)
                  + b_ref[...])


def kernel(x, edge_index, templates, templates_features, W, b):
    n, f = x.shape
    e = edge_index.shape[1]
    nk, nj = templates_features.shape[0], templates_features.shape[1]

    tfm = jnp.transpose(templates_features, (1, 0, 2)).reshape(nj * nk, f).T
    tmat = templates.reshape(nk, -1).T
    wt = W.T
    b1 = b.reshape(1, -1)

    blk = 2000
    grid1 = pl.cdiv(n, blk)
    table = pl.pallas_call(
        functools.partial(_feat_body, nk=nk, nj=nj),
        grid=(grid1,),
        in_specs=[
            pl.BlockSpec((blk, f), lambda i: (i, 0)),
            pl.BlockSpec((f, nj * nk), lambda i: (0, 0)),
        ],
        out_specs=pl.BlockSpec((blk, D), lambda i: (i, 0)),
        out_shape=jax.ShapeDtypeStruct((n, D), jnp.float32),
    )(x, tfm)

    rows_needed = pl.cdiv(e, EPR)
    rpt = pl.cdiv(rows_needed, NW * UNROLL) * UNROLL
    e_pad = rpt * NW * EPR
    src = jnp.concatenate(
        [edge_index[0], jnp.zeros((e_pad - e,), jnp.int32)])
    dst = jnp.concatenate(
        [edge_index[1], jnp.full((e_pad - e,), n, jnp.int32)])
    src2d = src.reshape(e_pad // EPR, EPR)
    dst2d = dst.reshape(e_pad // EPR, EPR)

    s_rpt = (n // NS // ZCH + 1) * ZCH
    n_pad = NS * s_rpt
    zeros_h = jnp.zeros((ZCH, D), jnp.float32)

    mesh = plsc.VectorSubcoreMesh(core_axis_name="c", subcore_axis_name="s")
    parts = pl.kernel(
        functools.partial(_sc_body, rpt=rpt, s_rpt=s_rpt),
        out_type=jax.ShapeDtypeStruct((NC, n_pad, D), jnp.float32),
        mesh=mesh,
        scratch_types=[
            pltpu.VMEM((UNROLL, EPR), jnp.int32),
            pltpu.VMEM((UNROLL, EPR), jnp.int32),
            pltpu.VMEM((EPR, D), jnp.float32),
            pltpu.VMEM((ZCH, D), jnp.float32),
            pltpu.VMEM_SHARED((n_pad, D), jnp.float32),
            pltpu.SemaphoreType.DMA,
        ],
    )(src2d, dst2d, table, zeros_h)

    grid3 = pl.cdiv(n, blk)
    out = pl.pallas_call(
        functools.partial(_final_body, nk=nk),
        grid=(grid3,),
        in_specs=[
            pl.BlockSpec((blk, D), lambda i: (i, 0)),
            pl.BlockSpec((NC, blk, D), lambda i: (0, i, 0)),
            pl.BlockSpec(tmat.shape, lambda i: (0, 0)),
            pl.BlockSpec(wt.shape, lambda i: (0, 0)),
            pl.BlockSpec((1, b1.shape[1]), lambda i: (0, 0)),
        ],
        out_specs=pl.BlockSpec((blk, W.shape[0]), lambda i: (i, 0)),
        out_shape=jax.ShapeDtypeStruct((n, W.shape[0]), jnp.float32),
    )(table, parts, tmat, wt, b1)
    return out


# speedup vs baseline: 13.6967x; 13.6967x over previous
import functools

import jax
import jax.numpy as jnp
from jax import lax
from jax.experimental import pallas as pl
from jax.experimental.pallas import tpu as pltpu
from jax.experimental.pallas import tpu_sc as plsc

D = 16
NC = 2
NS = 16
NW = NC * NS
EPR = 128
UNROLL = 8
ZCH = 800


def _feat_body(x_ref, tfm_ref, o_ref, *, nk, nj):
    xb = x_ref[...]
    tfm = tfm_ref[...]
    tsq = jnp.sum(tfm * tfm, axis=0, keepdims=True)
    y = tsq - 2.0 * jnp.dot(xb, tfm, preferred_element_type=jnp.float32)
    m = y[:, 0:nk]
    for j in range(1, nj):
        m = jnp.minimum(m, y[:, j * nk:(j + 1) * nk])
    fd = m + jnp.sum(xb * xb, axis=1, keepdims=True)
    bsz = fd.shape[0]
    ones = jnp.ones((bsz, 1), jnp.float32)
    zeros = jnp.zeros((bsz, D - nk - 1), jnp.float32)
    o_ref[...] = jnp.concatenate([fd, ones, zeros], axis=1)


def _sc_body(src_hbm, dst_hbm, table_hbm, zeros_hbm, out_hbm,
             src_v, dst_v, rows_v, zbuf_v, agg_sh, sem,
             *, rpt, s_rpt):
    c = lax.axis_index("c")
    s = lax.axis_index("s")
    wid = c * NS + s

    pltpu.sync_copy(zeros_hbm, zbuf_v)
    my0 = s * s_rpt

    def _zero(k, _):
        pltpu.sync_copy(zbuf_v, agg_sh.at[pl.ds(my0 + k * ZCH, ZCH)])
        return 0

    lax.fori_loop(0, s_rpt // ZCH, _zero, 0)
    plsc.subcore_barrier()

    row0 = wid * rpt

    def _step(g, _):
        base = row0 + g * UNROLL
        pltpu.sync_copy(src_hbm.at[pl.ds(base, UNROLL)], src_v)
        pltpu.sync_copy(dst_hbm.at[pl.ds(base, UNROLL)], dst_v)
        for j in range(UNROLL):
            pltpu.async_copy(table_hbm.at[src_v.at[j]], rows_v, sem).wait()
            pltpu.sync_copy(rows_v, agg_sh.at[dst_v.at[j]], add=True)
        return 0

    lax.fori_loop(0, rpt // UNROLL, _step, 0)
    plsc.subcore_barrier()

    def _emit(k, _):
        r = my0 + k * ZCH
        pltpu.sync_copy(agg_sh.at[pl.ds(r, ZCH)], zbuf_v)
        pltpu.sync_copy(zbuf_v, out_hbm.at[c, pl.ds(r, ZCH)])
        return 0

    lax.fori_loop(0, s_rpt // ZCH, _emit, 0)


def _final_body(tab_ref, p_ref, tmat_ref, wt_ref, b_ref, o_ref, *, nk):
    a = p_ref[0] + p_ref[1]
    deg = jnp.maximum(a[:, nk:nk + 1], 1.0)
    nm = a[:, 0:nk] / deg
    fd = tab_ref[...][:, 0:nk]
    tmat = tmat_ref[...]
    ones = jnp.ones((1, tmat.shape[0]), jnp.float32)
    struct = jnp.dot(ones, tmat, preferred_element_type=jnp.float32)
    struct = struct * (1.0 / tmat.shape[0])
    h = 0.5 * (fd + nm) + struct
    o_ref[...] = (jnp.dot(h, wt_ref[...], preferred_element_type=jnp.float32)
                  + b_ref[...])


def kernel(x, edge_index, templates, templates_features, W, b):
    n, f = x.shape
    e = edge_index.shape[1]
    nk, nj = templates_features.shape[0], templates_features.shape[1]

    tfm = jnp.transpose(templates_features, (1, 0, 2)).reshape(nj * nk, f).T
    tmat = templates.reshape(nk, -1).T
    wt = W.T
    b1 = b.reshape(1, -1)

    blk = 2000
    grid1 = pl.cdiv(n, blk)
    table = pl.pallas_call(
        functools.partial(_feat_body, nk=nk, nj=nj),
        grid=(grid1,),
        in_specs=[
            pl.BlockSpec((blk, f), lambda i: (i, 0)),
            pl.BlockSpec((f, nj * nk), lambda i: (0, 0)),
        ],
        out_specs=pl.BlockSpec((blk, D), lambda i: (i, 0)),
        out_shape=jax.ShapeDtypeStruct((n, D), jnp.float32),
    )(x, tfm)

    rows_needed = pl.cdiv(e, EPR)
    rpt = pl.cdiv(rows_needed, NW * UNROLL) * UNROLL
    e_pad = rpt * NW * EPR
    src = jnp.concatenate(
        [edge_index[0], jnp.zeros((e_pad - e,), jnp.int32)])
    dst = jnp.concatenate(
        [edge_index[1], jnp.full((e_pad - e,), n, jnp.int32)])
    src2d = src.reshape(e_pad // EPR, EPR)
    dst2d = dst.reshape(e_pad // EPR, EPR)

    s_rpt = (n // NS // ZCH + 1) * ZCH
    n_pad = NS * s_rpt
    zeros_h = jnp.zeros((ZCH, D), jnp.float32)

    mesh = plsc.VectorSubcoreMesh(core_axis_name="c", subcore_axis_name="s")
    parts = pl.kernel(
        functools.partial(_sc_body, rpt=rpt, s_rpt=s_rpt),
        out_type=jax.ShapeDtypeStruct((NC, n_pad, D), jnp.float32),
        mesh=mesh,
        scratch_types=[
            pltpu.VMEM((UNROLL, EPR), jnp.int32),
            pltpu.VMEM((UNROLL, EPR), jnp.int32),
            pltpu.VMEM((EPR, D), jnp.float32),
            pltpu.VMEM((ZCH, D), jnp.float32),
            pltpu.VMEM_SHARED((n_pad, D), jnp.float32),
            pltpu.SemaphoreType.DMA,
        ],
        compiler_params=pltpu.CompilerParams(use_tc_tiling_on_sc=False),
    )(src2d, dst2d, table, zeros_h)

    grid3 = pl.cdiv(n, blk)
    out = pl.pallas_call(
        functools.partial(_final_body, nk=nk),
        grid=(grid3,),
        in_specs=[
            pl.BlockSpec((blk, D), lambda i: (i, 0)),
            pl.BlockSpec((NC, blk, D), lambda i: (0, i, 0)),
            pl.BlockSpec(tmat.shape, lambda i: (0, 0)),
            pl.BlockSpec(wt.shape, lambda i: (0, 0)),
            pl.BlockSpec((1, b1.shape[1]), lambda i: (0, 0)),
        ],
        out_specs=pl.BlockSpec((blk, W.shape[0]), lambda i: (i, 0)),
        out_shape=jax.ShapeDtypeStruct((n, W.shape[0]), jnp.float32),
    )(table, parts, tmat, wt, b1)
    return out


# speedup vs baseline: 19.0554x; 1.3912x over previous
import functools

import jax
import jax.numpy as jnp
from jax import lax
from jax.experimental import pallas as pl
from jax.experimental.pallas import tpu as pltpu
from jax.experimental.pallas import tpu_sc as plsc

D = 16
NC = 2
NS = 16
NW = NC * NS
EPR = 128
U = 10
ZCH = 800


def _feat_body(x_ref, tfm_ref, o_ref, *, nk, nj):
    xb = x_ref[...]
    tfm = tfm_ref[...]
    tsq = jnp.sum(tfm * tfm, axis=0, keepdims=True)
    y = tsq - 2.0 * jnp.dot(xb, tfm, preferred_element_type=jnp.float32)
    m = y[:, 0:nk]
    for j in range(1, nj):
        m = jnp.minimum(m, y[:, j * nk:(j + 1) * nk])
    fd = m + jnp.sum(xb * xb, axis=1, keepdims=True)
    bsz = fd.shape[0]
    ones = jnp.ones((bsz, 1), jnp.float32)
    zeros = jnp.zeros((bsz, D - nk - 1), jnp.float32)
    o_ref[...] = jnp.concatenate([fd, ones, zeros], axis=1)


def _sc_body(edges_hbm, table_hbm, zeros_hbm, out_hbm,
             src_v, dst_v, rows_v, zbuf_v, agg_sh, gsem, ssem,
             *, n_rows, s_rpt):
    c = lax.axis_index("c")
    s = lax.axis_index("s")
    wid = c * NS + s

    pltpu.sync_copy(zeros_hbm, zbuf_v)
    my0 = s * s_rpt

    def _zero(k, _):
        pltpu.sync_copy(zbuf_v, agg_sh.at[pl.ds(my0 + k * ZCH, ZCH)])
        return 0

    lax.fori_loop(0, s_rpt // ZCH, _zero, 0)
    plsc.subcore_barrier()

    base = n_rows // NW
    rem = n_rows % NW
    cnt = base + jnp.where(wid < rem, 1, 0)
    start = wid * base + jnp.minimum(wid, rem)
    n_chunks = cnt // U

    def _chunk(g, _):
        r0 = start + g * U
        pltpu.sync_copy(edges_hbm.at[0, pl.ds(r0, U)], src_v)
        pltpu.sync_copy(edges_hbm.at[1, pl.ds(r0, U)], dst_v)
        gat = {0: pltpu.async_copy(table_hbm.at[src_v.at[0]],
                                   rows_v.at[0], gsem)}
        sca = {}
        for j in range(U):
            if j + 1 < U:
                if j - 1 >= 0:
                    sca[j - 1].wait()
                gat[j + 1] = pltpu.async_copy(
                    table_hbm.at[src_v.at[j + 1]],
                    rows_v.at[(j + 1) % 2], gsem)
            gat[j].wait()
            sca[j] = pltpu.async_copy(
                rows_v.at[j % 2], agg_sh.at[dst_v.at[j]], ssem, add=True)
        sca[U - 2].wait()
        sca[U - 1].wait()
        return 0

    lax.fori_loop(0, n_chunks, _chunk, 0)

    def _tail(t, _):
        r = start + n_chunks * U + t
        pltpu.sync_copy(edges_hbm.at[0, pl.ds(r, 1)], src_v.at[pl.ds(0, 1)])
        pltpu.sync_copy(edges_hbm.at[1, pl.ds(r, 1)], dst_v.at[pl.ds(0, 1)])
        pltpu.async_copy(table_hbm.at[src_v.at[0]], rows_v.at[0], gsem).wait()
        pltpu.sync_copy(rows_v.at[0], agg_sh.at[dst_v.at[0]], add=True)
        return 0

    lax.fori_loop(0, cnt - n_chunks * U, _tail, 0)
    plsc.subcore_barrier()

    def _emit(k, _):
        r = my0 + k * ZCH
        pltpu.sync_copy(agg_sh.at[pl.ds(r, ZCH)], zbuf_v)
        pltpu.sync_copy(zbuf_v, out_hbm.at[c, pl.ds(r, ZCH)])
        return 0

    lax.fori_loop(0, s_rpt // ZCH, _emit, 0)


def _final_body(tab_ref, p_ref, tmat_ref, wt_ref, b_ref, o_ref, *, nk):
    a = p_ref[0] + p_ref[1]
    deg = jnp.maximum(a[:, nk:nk + 1], 1.0)
    nm = a[:, 0:nk] / deg
    fd = tab_ref[...][:, 0:nk]
    tmat = tmat_ref[...]
    ones = jnp.ones((1, tmat.shape[0]), jnp.float32)
    struct = jnp.dot(ones, tmat, preferred_element_type=jnp.float32)
    struct = struct * (1.0 / tmat.shape[0])
    h = 0.5 * (fd + nm) + struct
    o_ref[...] = (jnp.dot(h, wt_ref[...], preferred_element_type=jnp.float32)
                  + b_ref[...])


def kernel(x, edge_index, templates, templates_features, W, b):
    n, f = x.shape
    e = edge_index.shape[1]
    nk, nj = templates_features.shape[0], templates_features.shape[1]

    tfm = jnp.transpose(templates_features, (1, 0, 2)).reshape(nj * nk, f).T
    tmat = templates.reshape(nk, -1).T
    wt = W.T
    b1 = b.reshape(1, -1)

    blk = 4000
    grid1 = pl.cdiv(n, blk)
    table = pl.pallas_call(
        functools.partial(_feat_body, nk=nk, nj=nj),
        grid=(grid1,),
        in_specs=[
            pl.BlockSpec((blk, f), lambda i: (i, 0)),
            pl.BlockSpec((f, nj * nk), lambda i: (0, 0)),
        ],
        out_specs=pl.BlockSpec((blk, D), lambda i: (i, 0)),
        out_shape=jax.ShapeDtypeStruct((n, D), jnp.float32),
    )(x, tfm)

    if e % EPR:
        pad = EPR - e % EPR
        edge_index = jnp.concatenate(
            [edge_index,
             jnp.concatenate([jnp.zeros((1, pad), jnp.int32),
                              jnp.full((1, pad), n, jnp.int32)])], axis=1)
    n_rows = edge_index.shape[1] // EPR
    e3 = edge_index.reshape(2, n_rows, EPR)

    s_rpt = (n // NS // ZCH + 1) * ZCH
    n_pad = NS * s_rpt
    zeros_h = jnp.zeros((ZCH, D), jnp.float32)

    mesh = plsc.VectorSubcoreMesh(core_axis_name="c", subcore_axis_name="s")
    parts = pl.kernel(
        functools.partial(_sc_body, n_rows=n_rows, s_rpt=s_rpt),
        out_type=jax.ShapeDtypeStruct((NC, n_pad, D), jnp.float32),
        mesh=mesh,
        scratch_types=[
            pltpu.VMEM((U, EPR), jnp.int32),
            pltpu.VMEM((U, EPR), jnp.int32),
            pltpu.VMEM((2, EPR, D), jnp.float32),
            pltpu.VMEM((ZCH, D), jnp.float32),
            pltpu.VMEM_SHARED((n_pad, D), jnp.float32),
            pltpu.SemaphoreType.DMA,
            pltpu.SemaphoreType.DMA,
        ],
        compiler_params=pltpu.CompilerParams(use_tc_tiling_on_sc=False),
    )(e3, table, zeros_h)

    grid3 = pl.cdiv(n, blk)
    out = pl.pallas_call(
        functools.partial(_final_body, nk=nk),
        grid=(grid3,),
        in_specs=[
            pl.BlockSpec((blk, D), lambda i: (i, 0)),
            pl.BlockSpec((NC, blk, D), lambda i: (0, i, 0)),
            pl.BlockSpec(tmat.shape, lambda i: (0, 0)),
            pl.BlockSpec(wt.shape, lambda i: (0, 0)),
            pl.BlockSpec((1, b1.shape[1]), lambda i: (0, 0)),
        ],
        out_specs=pl.BlockSpec((blk, W.shape[0]), lambda i: (i, 0)),
        out_shape=jax.ShapeDtypeStruct((n, W.shape[0]), jnp.float32),
    )(table, parts, tmat, wt, b1)
    return out


# speedup vs baseline: 24.8586x; 1.3045x over previous
import functools

import jax
import jax.numpy as jnp
from jax import lax
from jax.experimental import pallas as pl
from jax.experimental.pallas import tpu as pltpu
from jax.experimental.pallas import tpu_sc as plsc

D = 16
G = 8
FS = 8
NC = 2
NS = 16
NW = NC * NS
EPR = 128
U = 10
ZCH = 800


def _feat_body(xp_ref, wj_ref, s_ref, o_ref, *, nj, nk):
    xp = xp_ref[...]
    sdot = jnp.dot(xp * xp, s_ref[...],
                   preferred_element_type=jnp.float32)
    w = wj_ref[...]
    m = jnp.dot(xp, w[0:G * FS], preferred_element_type=jnp.float32)
    for j in range(1, nj):
        m = jnp.minimum(
            m, jnp.dot(xp, w[j * G * FS:(j + 1) * G * FS],
                       preferred_element_type=jnp.float32))
    fd = m + sdot
    lane = lax.broadcasted_iota(jnp.int32, (1, 128), 1) % D
    o_ref[...] = jnp.where(lane == nk, 1.0,
                           jnp.where(lane < nk, fd, 0.0))


def _sc_body(edges_hbm, table_hbm, zeros_hbm, out_hbm,
             src_v, dst_v, rows_v, zbuf_v, agg_sh, gsem, ssem,
             *, n_rows, s_rpt):
    c = lax.axis_index("c")
    s = lax.axis_index("s")
    wid = c * NS + s

    pltpu.sync_copy(zeros_hbm, zbuf_v)
    my0 = s * s_rpt

    def _zero(k, _):
        pltpu.sync_copy(zbuf_v, agg_sh.at[pl.ds(my0 + k * ZCH, ZCH)])
        return 0

    lax.fori_loop(0, s_rpt // ZCH, _zero, 0)
    plsc.subcore_barrier()

    base = n_rows // NW
    rem = n_rows % NW
    cnt = base + jnp.where(wid < rem, 1, 0)
    start = wid * base + jnp.minimum(wid, rem)
    n_chunks = cnt // U

    def _chunk(g, _):
        r0 = start + g * U
        pltpu.sync_copy(edges_hbm.at[0, pl.ds(r0, U)], src_v)
        pltpu.sync_copy(edges_hbm.at[1, pl.ds(r0, U)], dst_v)
        gat = {0: pltpu.async_copy(table_hbm.at[src_v.at[0]],
                                   rows_v.at[0], gsem)}
        sca = {}
        for j in range(U):
            if j + 1 < U:
                if j - 1 >= 0:
                    sca[j - 1].wait()
                gat[j + 1] = pltpu.async_copy(
                    table_hbm.at[src_v.at[j + 1]],
                    rows_v.at[(j + 1) % 2], gsem)
            gat[j].wait()
            sca[j] = pltpu.async_copy(
                rows_v.at[j % 2], agg_sh.at[dst_v.at[j]], ssem, add=True)
        sca[U - 2].wait()
        sca[U - 1].wait()
        return 0

    lax.fori_loop(0, n_chunks, _chunk, 0)

    def _tail(t, _):
        r = start + n_chunks * U + t
        pltpu.sync_copy(edges_hbm.at[0, pl.ds(r, 1)], src_v.at[pl.ds(0, 1)])
        pltpu.sync_copy(edges_hbm.at[1, pl.ds(r, 1)], dst_v.at[pl.ds(0, 1)])
        pltpu.async_copy(table_hbm.at[src_v.at[0]], rows_v.at[0], gsem).wait()
        pltpu.sync_copy(rows_v.at[0], agg_sh.at[dst_v.at[0]], add=True)
        return 0

    lax.fori_loop(0, cnt - n_chunks * U, _tail, 0)
    plsc.subcore_barrier()

    def _emit(k, _):
        r = my0 + k * ZCH
        pltpu.sync_copy(agg_sh.at[pl.ds(r, ZCH)], zbuf_v)
        pltpu.sync_copy(zbuf_v, out_hbm.at[c, pl.ds(r, ZCH)])
        return 0

    lax.fori_loop(0, s_rpt // ZCH, _emit, 0)


def _final_body(tab_ref, p_ref, sel_ref, wtb_ref, st_ref, bp_ref, o_ref):
    a = p_ref[0] + p_ref[1]
    degb = jnp.dot(a, sel_ref[...],
                   preferred_element_type=jnp.float32)
    inv = 1.0 / jnp.maximum(degb, 1.0)
    h = 0.5 * (tab_ref[...] + a * inv) + st_ref[...]
    o_ref[...] = (jnp.dot(h, wtb_ref[...], preferred_element_type=jnp.float32)
                  + bp_ref[...])


def kernel(x, edge_index, templates, templates_features, W, b):
    n, f = x.shape
    e = edge_index.shape[1]
    nk, nj = templates_features.shape[0], templates_features.shape[1]
    nc = W.shape[0]
    s_rpt = (n // NS // ZCH + 1) * ZCH
    n_pad = NS * s_rpt
    npk = n_pad // G

    xg = jnp.concatenate([x, jnp.zeros((n_pad - n, f), x.dtype)])
    xpp = jnp.concatenate(
        [xg.reshape(npk, G, f),
         jnp.ones((npk, G, 1), jnp.float32),
         jnp.zeros((npk, G, FS - f - 1), jnp.float32)],
        axis=2).reshape(npk, G * FS)

    tf = templates_features
    tsq = jnp.sum(tf * tf, axis=2)
    w_small = jnp.zeros((nj, FS, D), jnp.float32)
    w_small = w_small.at[:, 0:f, 0:nk].set(-2.0 * jnp.transpose(tf, (1, 2, 0)))
    w_small = w_small.at[:, f, 0:nk].set(tsq.T - 1.0)
    eye = jnp.eye(G, dtype=jnp.float32)
    wj_big = jnp.stack(
        [jnp.kron(eye, w_small[j]) for j in range(nj)]
    ).reshape(nj * G * FS, G * D)
    s_big = jnp.kron(eye, jnp.ones((FS, D), jnp.float32))

    sel_small = jnp.zeros((D, D), jnp.float32).at[nk, :].set(1.0)
    sel_big = jnp.kron(eye, sel_small)
    wt_small = jnp.zeros((D, D), jnp.float32).at[0:nk, 0:nc].set(W.T)
    wt_big = jnp.kron(eye, wt_small)
    struct = jnp.mean(templates, axis=(1, 2))
    st_p = jnp.tile(jnp.pad(struct, (0, D - nk)), G).reshape(1, G * D)
    b_p = jnp.tile(jnp.pad(b, (0, D - nc)), G).reshape(1, G * D)

    blk = 3200
    grid1 = pl.cdiv(npk, blk)
    table_pk = pl.pallas_call(
        functools.partial(_feat_body, nj=nj, nk=nk),
        grid=(grid1,),
        in_specs=[
            pl.BlockSpec((blk, G * FS), lambda i: (i, 0)),
            pl.BlockSpec(wj_big.shape, lambda i: (0, 0)),
            pl.BlockSpec(s_big.shape, lambda i: (0, 0)),
        ],
        out_specs=pl.BlockSpec((blk, G * D), lambda i: (i, 0)),
        out_shape=jax.ShapeDtypeStruct((npk, G * D), jnp.float32),
    )(xpp, wj_big, s_big)
    table = table_pk.reshape(npk * G, D)

    if e % EPR:
        pad = EPR - e % EPR
        edge_index = jnp.concatenate(
            [edge_index,
             jnp.concatenate([jnp.zeros((1, pad), jnp.int32),
                              jnp.full((1, pad), n, jnp.int32)])], axis=1)
    n_rows = edge_index.shape[1] // EPR
    e3 = edge_index.reshape(2, n_rows, EPR)
    zeros_h = jnp.zeros((ZCH, D), jnp.float32)

    mesh = plsc.VectorSubcoreMesh(core_axis_name="c", subcore_axis_name="s")
    parts = pl.kernel(
        functools.partial(_sc_body, n_rows=n_rows, s_rpt=s_rpt),
        out_type=jax.ShapeDtypeStruct((NC, n_pad, D), jnp.float32),
        mesh=mesh,
        scratch_types=[
            pltpu.VMEM((U, EPR), jnp.int32),
            pltpu.VMEM((U, EPR), jnp.int32),
            pltpu.VMEM((2, EPR, D), jnp.float32),
            pltpu.VMEM((ZCH, D), jnp.float32),
            pltpu.VMEM_SHARED((n_pad, D), jnp.float32),
            pltpu.SemaphoreType.DMA,
            pltpu.SemaphoreType.DMA,
        ],
        compiler_params=pltpu.CompilerParams(use_tc_tiling_on_sc=False),
    )(e3, table, zeros_h)
    parts_pk = parts.reshape(NC, npk, G * D)

    grid3 = pl.cdiv(npk, blk)
    out_pk = pl.pallas_call(
        _final_body,
        grid=(grid3,),
        in_specs=[
            pl.BlockSpec((blk, G * D), lambda i: (i, 0)),
            pl.BlockSpec((NC, blk, G * D), lambda i: (0, i, 0)),
            pl.BlockSpec(sel_big.shape, lambda i: (0, 0)),
            pl.BlockSpec(wt_big.shape, lambda i: (0, 0)),
            pl.BlockSpec((1, G * D), lambda i: (0, 0)),
            pl.BlockSpec((1, G * D), lambda i: (0, 0)),
        ],
        out_specs=pl.BlockSpec((blk, G * D), lambda i: (i, 0)),
        out_shape=jax.ShapeDtypeStruct((npk, G * D), jnp.float32),
    )(table_pk, parts_pk, sel_big, wt_big, st_p, b_p)

    return out_pk.reshape(npk * G, D)[:n, :nc]


# speedup vs baseline: 28.8755x; 1.1616x over previous
import functools

import jax
import jax.numpy as jnp
from jax import lax
from jax.experimental import pallas as pl
from jax.experimental.pallas import tpu as pltpu
from jax.experimental.pallas import tpu_sc as plsc

D = 16
G = 8
FS = 8
NC = 2
NS = 16
NW = NC * NS
EPR = 128
U = 10
ZCH = 800


def _feat_body(xp_ref, wj_ref, s_ref, o_ref, *, nj, nk):
    xp = xp_ref[...]
    sdot = jnp.dot(xp * xp, s_ref[...],
                   preferred_element_type=jnp.float32)
    w = wj_ref[...]
    m = jnp.dot(xp, w[0:G * FS], preferred_element_type=jnp.float32)
    for j in range(1, nj):
        m = jnp.minimum(
            m, jnp.dot(xp, w[j * G * FS:(j + 1) * G * FS],
                       preferred_element_type=jnp.float32))
    fd = m + sdot
    lane = lax.broadcasted_iota(jnp.int32, (1, 128), 1) % D
    o_ref[...] = jnp.where(lane == nk, 1.0,
                           jnp.where(lane < nk, fd, 0.0))


def _sc_body(edges_hbm, table_hbm, zeros_hbm, out_hbm,
             src_v, dst_v, rows_v, zbuf_v, agg_sh, gsem, ssem,
             *, n_rows, s_rpt):
    c = lax.axis_index("c")
    s = lax.axis_index("s")
    wid = c * NS + s

    pltpu.sync_copy(zeros_hbm, zbuf_v)
    my0 = s * s_rpt

    def _zero(k, _):
        pltpu.sync_copy(zbuf_v, agg_sh.at[pl.ds(my0 + k * ZCH, ZCH)])
        return 0

    lax.fori_loop(0, s_rpt // ZCH, _zero, 0)
    plsc.subcore_barrier()

    base = n_rows // NW
    rem = n_rows % NW
    cnt = base + jnp.where(wid < rem, 1, 0)
    start = wid * base + jnp.minimum(wid, rem)
    n_chunks = cnt // U

    def _chunk(g, _):
        r0 = start + g * U
        pltpu.sync_copy(edges_hbm.at[0, pl.ds(r0, U)], src_v)
        pltpu.sync_copy(edges_hbm.at[1, pl.ds(r0, U)], dst_v)
        gat = {j: pltpu.async_copy(table_hbm.at[src_v.at[j]],
                                   rows_v.at[j], gsem) for j in range(3)}
        sca = {}
        for j in range(U):
            if j + 3 < U:
                if j - 1 >= 0:
                    sca[j - 1].wait()
                gat[j + 3] = pltpu.async_copy(
                    table_hbm.at[src_v.at[j + 3]],
                    rows_v.at[(j + 3) % 4], gsem)
            gat[j].wait()
            sca[j] = pltpu.async_copy(
                rows_v.at[j % 4], agg_sh.at[dst_v.at[j]], ssem, add=True)
        for j in range(max(0, U - 4), U):
            sca[j].wait()
        return 0

    lax.fori_loop(0, n_chunks, _chunk, 0)

    def _tail(t, _):
        r = start + n_chunks * U + t
        pltpu.sync_copy(edges_hbm.at[0, pl.ds(r, 1)], src_v.at[pl.ds(0, 1)])
        pltpu.sync_copy(edges_hbm.at[1, pl.ds(r, 1)], dst_v.at[pl.ds(0, 1)])
        pltpu.async_copy(table_hbm.at[src_v.at[0]], rows_v.at[0], gsem).wait()
        pltpu.sync_copy(rows_v.at[0], agg_sh.at[dst_v.at[0]], add=True)
        return 0

    lax.fori_loop(0, cnt - n_chunks * U, _tail, 0)
    plsc.subcore_barrier()

    def _emit(k, _):
        r = my0 + k * ZCH
        pltpu.sync_copy(agg_sh.at[pl.ds(r, ZCH)], zbuf_v)
        pltpu.sync_copy(zbuf_v, out_hbm.at[c, pl.ds(r, ZCH)])
        return 0

    lax.fori_loop(0, s_rpt // ZCH, _emit, 0)


def _final_body(tab_ref, p_ref, sel_ref, wtb_ref, st_ref, bp_ref, o_ref):
    a = p_ref[0] + p_ref[1]
    degb = jnp.dot(a, sel_ref[...],
                   preferred_element_type=jnp.float32)
    inv = 1.0 / jnp.maximum(degb, 1.0)
    h = 0.5 * (tab_ref[...] + a * inv) + st_ref[...]
    o_ref[...] = (jnp.dot(h, wtb_ref[...], preferred_element_type=jnp.float32)
                  + bp_ref[...])


def kernel(x, edge_index, templates, templates_features, W, b):
    n, f = x.shape
    e = edge_index.shape[1]
    nk, nj = templates_features.shape[0], templates_features.shape[1]
    nc = W.shape[0]
    s_rpt = (n // NS // ZCH + 1) * ZCH
    n_pad = NS * s_rpt
    npk = n_pad // G

    xg = jnp.concatenate([x, jnp.zeros((n_pad - n, f), x.dtype)])
    xpp = jnp.concatenate(
        [xg.reshape(npk, G, f),
         jnp.ones((npk, G, 1), jnp.float32),
         jnp.zeros((npk, G, FS - f - 1), jnp.float32)],
        axis=2).reshape(npk, G * FS)

    tf = templates_features
    tsq = jnp.sum(tf * tf, axis=2)
    w_small = jnp.zeros((nj, FS, D), jnp.float32)
    w_small = w_small.at[:, 0:f, 0:nk].set(-2.0 * jnp.transpose(tf, (1, 2, 0)))
    w_small = w_small.at[:, f, 0:nk].set(tsq.T - 1.0)
    eye = jnp.eye(G, dtype=jnp.float32)
    wj_big = jnp.stack(
        [jnp.kron(eye, w_small[j]) for j in range(nj)]
    ).reshape(nj * G * FS, G * D)
    s_big = jnp.kron(eye, jnp.ones((FS, D), jnp.float32))

    ow = 4
    sel_small = jnp.zeros((D, D), jnp.float32).at[nk, :].set(1.0)
    sel_big = jnp.kron(eye, sel_small)
    wt_small = jnp.zeros((D, ow), jnp.float32).at[0:nk, 0:nc].set(W.T)
    wt_big = jnp.kron(eye, wt_small)
    struct = jnp.mean(templates, axis=(1, 2))
    st_p = jnp.tile(jnp.pad(struct, (0, D - nk)), G).reshape(1, G * D)
    b_p = jnp.tile(jnp.pad(b, (0, ow - nc)), G).reshape(1, G * ow)

    blk = 3200
    grid1 = pl.cdiv(npk, blk)
    table_pk = pl.pallas_call(
        functools.partial(_feat_body, nj=nj, nk=nk),
        grid=(grid1,),
        in_specs=[
            pl.BlockSpec((blk, G * FS), lambda i: (i, 0)),
            pl.BlockSpec(wj_big.shape, lambda i: (0, 0)),
            pl.BlockSpec(s_big.shape, lambda i: (0, 0)),
        ],
        out_specs=pl.BlockSpec((blk, G * D), lambda i: (i, 0)),
        out_shape=jax.ShapeDtypeStruct((npk, G * D), jnp.float32),
    )(xpp, wj_big, s_big)
    table = table_pk.reshape(npk * G, D)

    if e % EPR:
        pad = EPR - e % EPR
        edge_index = jnp.concatenate(
            [edge_index,
             jnp.concatenate([jnp.zeros((1, pad), jnp.int32),
                              jnp.full((1, pad), n, jnp.int32)])], axis=1)
    n_rows = edge_index.shape[1] // EPR
    e3 = edge_index.reshape(2, n_rows, EPR)
    zeros_h = jnp.zeros((ZCH, D), jnp.float32)

    mesh = plsc.VectorSubcoreMesh(core_axis_name="c", subcore_axis_name="s")
    parts = pl.kernel(
        functools.partial(_sc_body, n_rows=n_rows, s_rpt=s_rpt),
        out_type=jax.ShapeDtypeStruct((NC, n_pad, D), jnp.float32),
        mesh=mesh,
        scratch_types=[
            pltpu.VMEM((U, EPR), jnp.int32),
            pltpu.VMEM((U, EPR), jnp.int32),
            pltpu.VMEM((4, EPR, D), jnp.float32),
            pltpu.VMEM((ZCH, D), jnp.float32),
            pltpu.VMEM_SHARED((n_pad, D), jnp.float32),
            pltpu.SemaphoreType.DMA,
            pltpu.SemaphoreType.DMA,
        ],
        compiler_params=pltpu.CompilerParams(use_tc_tiling_on_sc=False),
    )(e3, table, zeros_h)
    parts_pk = parts.reshape(NC, npk, G * D)

    grid3 = pl.cdiv(npk, blk)
    out_pk = pl.pallas_call(
        _final_body,
        grid=(grid3,),
        in_specs=[
            pl.BlockSpec((blk, G * D), lambda i: (i, 0)),
            pl.BlockSpec((NC, blk, G * D), lambda i: (0, i, 0)),
            pl.BlockSpec(sel_big.shape, lambda i: (0, 0)),
            pl.BlockSpec(wt_big.shape, lambda i: (0, 0)),
            pl.BlockSpec((1, G * D), lambda i: (0, 0)),
            pl.BlockSpec((1, G * ow), lambda i: (0, 0)),
        ],
        out_specs=pl.BlockSpec((blk, G * ow), lambda i: (i, 0)),
        out_shape=jax.ShapeDtypeStruct((npk, G * ow), jnp.float32),
    )(table_pk, parts_pk, sel_big, wt_big, st_p, b_p)

    return out_pk.reshape(npk * G, ow)[:n, :nc]


# speedup vs baseline: 39.2260x; 1.3585x over previous
import functools

import jax
import jax.numpy as jnp
from jax import lax
from jax.experimental import pallas as pl
from jax.experimental.pallas import tpu as pltpu
from jax.experimental.pallas import tpu_sc as plsc

D = 16
G = 8
FS = 8
NC = 2
NS = 16
NW = NC * NS
EPR = 128
U = 10
NZC = 8


def _feat_body(xp_ref, wj_ref, s_ref, o_ref, *, nj, nk):
    xp = xp_ref[...]
    sdot = jnp.dot(xp * xp, s_ref[...],
                   preferred_element_type=jnp.float32)
    w = wj_ref[...]
    m = jnp.dot(xp, w[0:G * FS], preferred_element_type=jnp.float32)
    for j in range(1, nj):
        m = jnp.minimum(
            m, jnp.dot(xp, w[j * G * FS:(j + 1) * G * FS],
                       preferred_element_type=jnp.float32))
    fd = m + sdot
    lane = lax.broadcasted_iota(jnp.int32, (1, 128), 1) % D
    o_ref[...] = jnp.where(lane == nk, 1.0,
                           jnp.where(lane < nk, fd, 0.0))


def _sc_body(edges_hbm, table_hbm, zeros_hbm, out_hbm,
             src_v, dst_v, rows_v, zbuf_v, agg_sh, gsem, ssem, isem,
             *, n_rows, s_rpt):
    c = lax.axis_index("c")
    s = lax.axis_index("s")
    wid = c * NS + s

    pltpu.sync_copy(zeros_hbm, zbuf_v)
    my0 = s * s_rpt

    def _zero(k, _):
        zch = s_rpt // NZC
        pltpu.sync_copy(zbuf_v, agg_sh.at[pl.ds(my0 + k * zch, zch)])
        return 0

    lax.fori_loop(0, NZC, _zero, 0)
    plsc.subcore_barrier()

    base = n_rows // NW
    rem = n_rows % NW
    cnt = base + jnp.where(wid < rem, 1, 0)
    start = wid * base + jnp.minimum(wid, rem)
    n_chunks = cnt // U

    NB = 6

    def _idx_start(g, par):
        r0 = start + g * U
        pltpu.async_copy(edges_hbm.at[0, pl.ds(r0, U)], src_v.at[par], isem)
        pltpu.async_copy(edges_hbm.at[1, pl.ds(r0, U)], dst_v.at[par], isem)

    def _idx_wait(g, par):
        r0 = start + g * U
        pltpu.make_async_copy(
            edges_hbm.at[0, pl.ds(r0, U)], src_v.at[par], isem).wait()
        pltpu.make_async_copy(
            edges_hbm.at[1, pl.ds(r0, U)], dst_v.at[par], isem).wait()

    @pl.when(n_chunks > 0)
    def _prologue():
        _idx_start(0, 0)

    def _chunk(g, _):
        par = lax.rem(g, 2)
        _idx_wait(g, par)

        @pl.when(g + 1 < n_chunks)
        def _prefetch():
            _idx_start(g + 1, 1 - par)

        gat = {j: pltpu.async_copy(table_hbm.at[src_v.at[par, j]],
                                   rows_v.at[j], gsem)
               for j in range(NB - 1)}
        sca = {}
        for j in range(U):
            if j + NB - 1 < U:
                if j - 1 >= 0:
                    sca[j - 1].wait()
                gat[j + NB - 1] = pltpu.async_copy(
                    table_hbm.at[src_v.at[par, j + NB - 1]],
                    rows_v.at[(j + NB - 1) % NB], gsem)
            gat[j].wait()
            sca[j] = pltpu.async_copy(
                rows_v.at[j % NB], agg_sh.at[dst_v.at[par, j]], ssem,
                add=True)
        for j in range(max(0, U - NB), U):
            sca[j].wait()
        return 0

    lax.fori_loop(0, n_chunks, _chunk, 0)

    def _tail(t, _):
        r = start + n_chunks * U + t
        pltpu.sync_copy(edges_hbm.at[0, pl.ds(r, 1)],
                        src_v.at[0].at[pl.ds(0, 1)])
        pltpu.sync_copy(edges_hbm.at[1, pl.ds(r, 1)],
                        dst_v.at[0].at[pl.ds(0, 1)])
        pltpu.async_copy(table_hbm.at[src_v.at[0, 0]],
                         rows_v.at[0], gsem).wait()
        pltpu.sync_copy(rows_v.at[0], agg_sh.at[dst_v.at[0, 0]], add=True)
        return 0

    lax.fori_loop(0, cnt - n_chunks * U, _tail, 0)
    plsc.subcore_barrier()

    def _emit(k, _):
        zch = s_rpt // NZC
        r = my0 + k * zch
        pltpu.sync_copy(agg_sh.at[pl.ds(r, zch)], zbuf_v)
        pltpu.sync_copy(zbuf_v, out_hbm.at[c, pl.ds(r, zch)])
        return 0

    lax.fori_loop(0, NZC, _emit, 0)


def _final_body(tab_ref, p_ref, sel_ref, wtb_ref, st_ref, bp_ref, o_ref):
    a = p_ref[0] + p_ref[1]
    degb = jnp.dot(a, sel_ref[...],
                   preferred_element_type=jnp.float32)
    inv = 1.0 / jnp.maximum(degb, 1.0)
    h = 0.5 * (tab_ref[...] + a * inv) + st_ref[...]
    o = (jnp.dot(h, wtb_ref[...], preferred_element_type=jnp.float32)
         + bp_ref[...])
    o_ref[...] = o[0:o_ref.shape[0]]


def kernel(x, edge_index, templates, templates_features, W, b):
    n, f = x.shape
    e = edge_index.shape[1]
    nk, nj = templates_features.shape[0], templates_features.shape[1]
    nc = W.shape[0]
    s_rpt = NZC * -(-(n // NS + 1) // NZC)
    n_pad = NS * s_rpt
    npk = n_pad // G

    xg = jnp.concatenate([x, jnp.zeros((n_pad - n, f), x.dtype)])
    xpp = jnp.concatenate(
        [xg.reshape(npk, G, f),
         jnp.ones((npk, G, 1), jnp.float32),
         jnp.zeros((npk, G, FS - f - 1), jnp.float32)],
        axis=2).reshape(npk, G * FS)

    tf = templates_features
    tsq = jnp.sum(tf * tf, axis=2)
    w_small = jnp.zeros((nj, FS, D), jnp.float32)
    w_small = w_small.at[:, 0:f, 0:nk].set(-2.0 * jnp.transpose(tf, (1, 2, 0)))
    w_small = w_small.at[:, f, 0:nk].set(tsq.T - 1.0)
    eye = jnp.eye(G, dtype=jnp.float32)
    wj_big = jnp.stack(
        [jnp.kron(eye, w_small[j]) for j in range(nj)]
    ).reshape(nj * G * FS, G * D)
    s_big = jnp.kron(eye, jnp.ones((FS, D), jnp.float32))

    ow = 4
    sel_small = jnp.zeros((D, D), jnp.float32).at[nk, :].set(1.0)
    sel_big = jnp.kron(eye, sel_small)
    wt_small = jnp.zeros((D, ow), jnp.float32).at[0:nk, 0:nc].set(W.T)
    wt_big = jnp.kron(eye, wt_small)
    struct = jnp.mean(templates, axis=(1, 2))
    st_p = jnp.tile(jnp.pad(struct, (0, D - nk)), G).reshape(1, G * D)
    b_p = jnp.tile(jnp.pad(b, (0, ow - nc)), G).reshape(1, G * ow)

    blk = 3200
    grid1 = pl.cdiv(npk, blk)
    table_pk = pl.pallas_call(
        functools.partial(_feat_body, nj=nj, nk=nk),
        grid=(grid1,),
        in_specs=[
            pl.BlockSpec((blk, G * FS), lambda i: (i, 0)),
            pl.BlockSpec(wj_big.shape, lambda i: (0, 0)),
            pl.BlockSpec(s_big.shape, lambda i: (0, 0)),
        ],
        out_specs=pl.BlockSpec((blk, G * D), lambda i: (i, 0)),
        out_shape=jax.ShapeDtypeStruct((npk, G * D), jnp.float32),
    )(xpp, wj_big, s_big)
    table = table_pk.reshape(npk * G, D)

    if e % EPR:
        pad = EPR - e % EPR
        edge_index = jnp.concatenate(
            [edge_index,
             jnp.concatenate([jnp.zeros((1, pad), jnp.int32),
                              jnp.full((1, pad), n, jnp.int32)])], axis=1)
    n_rows = edge_index.shape[1] // EPR
    e3 = edge_index.reshape(2, n_rows, EPR)
    zeros_h = jnp.zeros((s_rpt // NZC, D), jnp.float32)

    mesh = plsc.VectorSubcoreMesh(core_axis_name="c", subcore_axis_name="s")
    parts = pl.kernel(
        functools.partial(_sc_body, n_rows=n_rows, s_rpt=s_rpt),
        out_type=jax.ShapeDtypeStruct((NC, n_pad, D), jnp.float32),
        mesh=mesh,
        scratch_types=[
            pltpu.VMEM((2, U, EPR), jnp.int32),
            pltpu.VMEM((2, U, EPR), jnp.int32),
            pltpu.VMEM((6, EPR, D), jnp.float32),
            pltpu.VMEM((s_rpt // NZC, D), jnp.float32),
            pltpu.VMEM_SHARED((n_pad, D), jnp.float32),
            pltpu.SemaphoreType.DMA,
            pltpu.SemaphoreType.DMA,
            pltpu.SemaphoreType.DMA,
        ],
        compiler_params=pltpu.CompilerParams(use_tc_tiling_on_sc=False),
    )(e3, table, zeros_h)
    parts_pk = parts.reshape(NC, npk, G * D)

    npo = n // G if n % G == 0 else npk
    out_pk = pl.pallas_call(
        _final_body,
        grid=(1,),
        in_specs=[
            pl.BlockSpec((npk, G * D), lambda i: (0, 0)),
            pl.BlockSpec((NC, npk, G * D), lambda i: (0, 0, 0)),
            pl.BlockSpec(sel_big.shape, lambda i: (0, 0)),
            pl.BlockSpec(wt_big.shape, lambda i: (0, 0)),
            pl.BlockSpec((1, G * D), lambda i: (0, 0)),
            pl.BlockSpec((1, G * ow), lambda i: (0, 0)),
        ],
        out_specs=pl.BlockSpec((npo, G * ow), lambda i: (0, 0)),
        out_shape=jax.ShapeDtypeStruct((npo, G * ow), jnp.float32),
    )(table_pk, parts_pk, sel_big, wt_big, st_p, b_p)

    return out_pk.reshape(npo * G, ow)[:n, :nc]
